# Initial kernel scaffold; baseline (speedup 1.0000x reference)
import jax
import jax.numpy as jnp
from jax.experimental import pallas as pl


def kernel(cnn_output, ca_W, ca_a_src, ca_a_dst, sa_W, sa_a_src, sa_a_dst, ca_edge_index, sa_edge_index):
    raise NotImplementedError("write your pallas kernel here")


# speedup vs baseline: 59.3442x; 59.3442x over previous
import functools

import jax
import jax.numpy as jnp
from jax import lax
from jax.experimental import pallas as pl
from jax.experimental.pallas import tpu as pltpu
from jax.experimental.pallas import tpu_sc as plsc

_SIZE = 7
_SP = 8
_L = 16


def _sum_hw_kernel(x_ref, o_ref):
    o_ref[0, 0, 0, :] = jnp.sum(x_ref[...], axis=(0, 2, 3))


def _lrelu(x):
    return jnp.where(x >= 0, x, 0.2 * x)


def _ca_gat_sc_body(sums_hbm, w_hbm, asrc_hbm, adst_hbm, out_hbm,
                    sums_v, w_v, as_v, ad_v, h_v, hs_v, hd_v, score_v,
                    *, n_b, n_c, n_heads, n_pix):
    wid = lax.axis_index("s") * 2 + lax.axis_index("c")
    n_chunks = n_c // _L
    n_sv = n_c // _L

    @pl.when(wid < n_chunks)
    def _():
        pltpu.sync_copy(sums_hbm, sums_v)
        pltpu.sync_copy(w_hbm, w_v)
        pltpu.sync_copy(asrc_hbm, as_v)
        pltpu.sync_copy(adst_hbm, ad_v)
        base = wid * _L
        inv_pix = 1.0 / float(n_pix)
        lane = lax.broadcasted_iota(jnp.int32, (_L,), 0)
        for b in range(n_b):
            acc = jnp.zeros((_L,), jnp.float32)
            for k in range(n_heads):
                wk = w_v[pl.ds(k * _L, _L)]
                ak_s = as_v[pl.ds(k * _L, _L)]
                ak_d = ad_v[pl.ds(k * _L, _L)]

                def build(i, pmax):
                    f = sums_v[b, pl.ds(i * _L, _L)] * inv_pix
                    h = f * wk
                    hs = h * ak_s
                    hd = h * ak_d
                    h_v[pl.ds(i * _L, _L)] = h
                    hs_v[pl.ds(i * _L, _L)] = hs
                    hd_v[pl.ds(i * _L, _L)] = hd
                    return jnp.maximum(pmax, hs)

                pmax = lax.fori_loop(0, n_sv, build,
                                     jnp.full((_L,), -1e30, jnp.float32))
                for sh in (1, 2, 4, 8):
                    perm = pmax.at[lane ^ sh].get(mode="promise_in_bounds")
                    pmax = jnp.maximum(pmax, perm)
                hd_vec = hd_v[pl.ds(base, _L)]
                m_vec = _lrelu(pmax + hd_vec)

                def srcloop(i, c):
                    den, num = c
                    hs_c = hs_v[pl.ds(i * _L, _L)]
                    h_c = h_v[pl.ds(i * _L, _L)]
                    for t in range(_L):
                        e = _lrelu(hs_c[t] + hd_vec)
                        ex = jnp.exp(e - m_vec)
                        den = den + ex
                        num = num + ex * h_c[t]
                    return den, num

                den, num = lax.fori_loop(
                    0, n_sv, srcloop,
                    (jnp.zeros((_L,), jnp.float32),
                     jnp.zeros((_L,), jnp.float32)))
                acc = acc + num / (den + 1e-16)
            score_v[...] = 1.0 / (1.0 + jnp.exp(-acc * (1.0 / n_heads)))
            pltpu.sync_copy(score_v, out_hbm.at[b, pl.ds(base, _L)])


def _sa_gat_sc_body(pp_hbm, jid_hbm, w_hbm, asrc_hbm, adst_hbm, out_hbm,
                    pp_v, sf_v, h_v, hs_v, w_v, as_v, ad_v, j_v, out_v,
                    *, n_b, n_cb, n_heads, norm, size):
    wid = lax.axis_index("s") * 2 + lax.axis_index("c")
    n_nodes = _SP * _SP
    n_ch = n_nodes // _L

    @pl.when(wid < n_b * n_ch)
    def _():
        b = wid // n_ch
        c4 = wid % n_ch
        pltpu.sync_copy(pp_hbm.at[b], pp_v)
        pltpu.sync_copy(jid_hbm.at[c4], j_v)
        pltpu.sync_copy(w_hbm, w_v)
        pltpu.sync_copy(asrc_hbm, as_v)
        pltpu.sync_copy(adst_hbm, ad_v)
        inv_norm = 1.0 / float(norm)
        for q in range(n_ch):
            def nsum(n, v):
                return v + pp_v[n, pl.ds(q * _L, _L)]
            s = lax.fori_loop(0, n_cb, nsum, jnp.zeros((_L,), jnp.float32))
            sf_v[pl.ds(q * _L, _L)] = s * inv_norm
        j = j_v[...]
        r = j >> 3
        cc = j & 7
        zeros = jnp.zeros((_L,), jnp.float32)
        h_v[pl.ds(0, _L)] = zeros
        h_v[pl.ds(_L + n_nodes, _L)] = zeros
        hs_v[pl.ds(0, _L)] = zeros
        hs_v[pl.ds(_L + n_nodes, _L)] = zeros
        offs = [(dr, dc) for dr in (-1, 0, 1) for dc in (-1, 0, 1)]
        acc = jnp.zeros((_L,), jnp.float32)
        for k in range(n_heads):
            wk = w_v[pl.ds(k * _L, _L)]
            ak_s = as_v[pl.ds(k * _L, _L)]
            ak_d = ad_v[pl.ds(k * _L, _L)]
            for q in range(n_ch):
                f = sf_v[pl.ds(q * _L, _L)]
                h = f * wk
                h_v[pl.ds(_L + q * _L, _L)] = h
                hs_v[pl.ds(_L + q * _L, _L)] = h * ak_s
            hd_vec = sf_v[pl.ds(c4 * _L, _L)] * wk * ak_d
            m = jnp.full((_L,), -1e30, jnp.float32)
            es = []
            for dr, dc in offs:
                rr = r + dr
                c2 = cc + dc
                valid = ((rr >= 0) & (rr < size) & (c2 >= 0) & (c2 < size)
                         & (r < size) & (cc < size))
                d = dr * _SP + dc
                hs_g = hs_v[pl.ds(_L + c4 * _L + d, _L)]
                e = _lrelu(hs_g + hd_vec)
                es.append((e, valid, d))
                m = jnp.maximum(m, jnp.where(valid, e, -1e30))
            den = jnp.zeros((_L,), jnp.float32)
            num = jnp.zeros((_L,), jnp.float32)
            for e, valid, d in es:
                ex = jnp.where(valid, jnp.exp(e - m), 0.0)
                den = den + ex
                num = num + ex * h_v[pl.ds(_L + c4 * _L + d, _L)]
            acc = acc + num / (den + 1e-16)
        out_v[...] = 1.0 / (1.0 + jnp.exp(-acc * (1.0 / n_heads)))
        pltpu.sync_copy(out_v, out_hbm.at[b, pl.ds(c4 * _L, _L)])


def _pool_kernel(x_ref, s_ref, o_ref, *, size, rblk):
    x = x_ref[0]
    s = s_ref[0, 0, 0, :]
    y = jnp.tanh(x * s[:, None, None])
    ysum = jnp.sum(y, axis=0)
    hw = ysum.shape[0]
    mr = (lax.broadcasted_iota(jnp.int32, (hw, _SP), 0) // rblk
          == lax.broadcasted_iota(jnp.int32, (hw, _SP), 1)).astype(jnp.float32)
    a = jnp.dot(ysum, mr, preferred_element_type=jnp.float32)
    ml = (lax.broadcasted_iota(jnp.int32, (_SP, hw), 1) // rblk
          == lax.broadcasted_iota(jnp.int32, (_SP, hw), 0)).astype(jnp.float32)
    o_ref[0, 0] = jnp.dot(ml, a, preferred_element_type=jnp.float32)


def _final_kernel(x_ref, s_ref, sa_ref, o_ref, *, size, rblk):
    x = x_ref[0]
    s = s_ref[0, 0, 0, :]
    y = jnp.tanh(x * s[:, None, None])
    smap = sa_ref[0]
    hw = x.shape[1]
    u = (lax.broadcasted_iota(jnp.int32, (hw, _SP), 0) // rblk
         == lax.broadcasted_iota(jnp.int32, (hw, _SP), 1)).astype(jnp.float32)
    ut = (lax.broadcasted_iota(jnp.int32, (_SP, hw), 1) // rblk
          == lax.broadcasted_iota(jnp.int32, (_SP, hw), 0)).astype(jnp.float32)
    t1 = jnp.dot(smap, ut, preferred_element_type=jnp.float32)
    scale = jnp.dot(u, t1, preferred_element_type=jnp.float32)
    o_ref[0] = x + y * scale


def kernel(cnn_output, ca_W, ca_a_src, ca_a_dst, sa_W, sa_a_src, sa_a_dst,
           ca_edge_index, sa_edge_index):
    B, C, H, W = cnn_output.shape
    n_heads = ca_W.shape[1]
    n_pix = H * W
    rblk = H // _SIZE
    CB = 32
    NCB = C // CB
    mesh = plsc.VectorSubcoreMesh(core_axis_name="c", subcore_axis_name="s",
                                  num_cores=2, num_subcores=16)

    def _tile(v):
        return jnp.broadcast_to(v.reshape(n_heads, 1), (n_heads, _L)).reshape(-1)

    ca_w_t = _tile(ca_W[0])
    ca_as_t = _tile(ca_a_src)
    ca_ad_t = _tile(ca_a_dst)
    sa_w_t = _tile(sa_W[0])
    sa_as_t = _tile(sa_a_src)
    sa_ad_t = _tile(sa_a_dst)

    sums4 = pl.pallas_call(
        _sum_hw_kernel,
        grid=(B, NCB),
        in_specs=[pl.BlockSpec((1, CB, H, W), lambda b, i: (b, i, 0, 0))],
        out_specs=pl.BlockSpec((1, 1, 1, CB), lambda b, i: (b, i, 0, 0)),
        out_shape=jax.ShapeDtypeStruct((B, NCB, 1, CB), jnp.float32),
    )(cnn_output)
    sums = sums4.reshape(B, C)

    ca_gat = functools.partial(
        pl.kernel,
        mesh=mesh,
        out_type=jax.ShapeDtypeStruct((B, C), jnp.float32),
        scratch_types=[
            pltpu.VMEM((B, C), jnp.float32),
            pltpu.VMEM((n_heads * _L,), jnp.float32),
            pltpu.VMEM((n_heads * _L,), jnp.float32),
            pltpu.VMEM((n_heads * _L,), jnp.float32),
            pltpu.VMEM((C,), jnp.float32),
            pltpu.VMEM((C,), jnp.float32),
            pltpu.VMEM((C,), jnp.float32),
            pltpu.VMEM((_L,), jnp.float32),
        ],
    )(functools.partial(_ca_gat_sc_body, n_b=B, n_c=C, n_heads=n_heads,
                        n_pix=n_pix))
    ca_scores = ca_gat(sums, ca_w_t, ca_as_t, ca_ad_t)
    scores4 = ca_scores.reshape(B, NCB, 1, CB)

    pooled_part = pl.pallas_call(
        functools.partial(_pool_kernel, size=_SIZE, rblk=rblk),
        grid=(B, NCB),
        in_specs=[
            pl.BlockSpec((1, CB, H, W), lambda b, i: (b, i, 0, 0)),
            pl.BlockSpec((1, 1, 1, CB), lambda b, i: (b, i, 0, 0)),
        ],
        out_specs=pl.BlockSpec((1, 1, _SP, _SP), lambda b, i: (b, i, 0, 0)),
        out_shape=jax.ShapeDtypeStruct((B, NCB, _SP, _SP), jnp.float32),
    )(cnn_output, scores4)
    pp = pooled_part.reshape(B, NCB, _SP * _SP)

    sa_gat = functools.partial(
        pl.kernel,
        mesh=mesh,
        out_type=jax.ShapeDtypeStruct((B, _SP * _SP), jnp.float32),
        scratch_types=[
            pltpu.VMEM((NCB, _SP * _SP), jnp.float32),
            pltpu.VMEM((_SP * _SP,), jnp.float32),
            pltpu.VMEM((2 * _L + _SP * _SP,), jnp.float32),
            pltpu.VMEM((2 * _L + _SP * _SP,), jnp.float32),
            pltpu.VMEM((n_heads * _L,), jnp.float32),
            pltpu.VMEM((n_heads * _L,), jnp.float32),
            pltpu.VMEM((n_heads * _L,), jnp.float32),
            pltpu.VMEM((_L,), jnp.int32),
            pltpu.VMEM((_L,), jnp.float32),
        ],
    )(functools.partial(_sa_gat_sc_body, n_b=B, n_cb=NCB, n_heads=n_heads,
                        norm=float(C * rblk * rblk), size=_SIZE))
    jid = jnp.arange(_SP * _SP, dtype=jnp.int32).reshape(-1, _L)
    sa_scores = sa_gat(pp, jid, sa_w_t, sa_as_t, sa_ad_t)
    sa3 = sa_scores.reshape(B, _SP, _SP)

    out = pl.pallas_call(
        functools.partial(_final_kernel, size=_SIZE, rblk=rblk),
        grid=(B, NCB),
        in_specs=[
            pl.BlockSpec((1, CB, H, W), lambda b, i: (b, i, 0, 0)),
            pl.BlockSpec((1, 1, 1, CB), lambda b, i: (b, i, 0, 0)),
            pl.BlockSpec((1, _SP, _SP), lambda b, i: (b, 0, 0)),
        ],
        out_specs=pl.BlockSpec((1, CB, H, W), lambda b, i: (b, i, 0, 0)),
        out_shape=jax.ShapeDtypeStruct((B, C, H, W), jnp.float32),
    )(cnn_output, scores4, sa3)
    return out
